# baseline (speedup 1.0000x reference)
import jax
import jax.numpy as jnp
from jax.experimental import pallas as pl
from jax.experimental.pallas import tpu as pltpu

SEQ = 4096
FEAT = 26
DIM = 64
BATCH = 4
ROWD = FEAT * DIM

BS = 512


def _add_body(x_ref, t_ref, o_ref):
    o_ref[...] = x_ref[...] + t_ref[...]


def kernel(inputs, table):
    x = inputs.reshape(BATCH, SEQ, ROWD)
    t = table.reshape(SEQ, ROWD)
    out = pl.pallas_call(
        _add_body,
        grid=(SEQ // BS, BATCH),
        in_specs=[
            pl.BlockSpec((1, BS, ROWD), lambda s, b: (b, s, 0)),
            pl.BlockSpec((BS, ROWD), lambda s, b: (s, 0)),
        ],
        out_specs=pl.BlockSpec((1, BS, ROWD), lambda s, b: (b, s, 0)),
        out_shape=jax.ShapeDtypeStruct((BATCH, SEQ, ROWD), jnp.float32),
        compiler_params=pltpu.CompilerParams(
            dimension_semantics=("arbitrary", "arbitrary"),
        ),
    )(x, t)
    return out.reshape(BATCH, SEQ, FEAT, DIM)
